# baseline (speedup 1.0000x reference)
import jax
import jax.numpy as jnp
from jax.experimental import pallas as pl
from jax.experimental.pallas import tpu as pltpu

_K = 2


def _body(x_ref, wg_ref, bg_ref, wn_ref, bn_ref, o_ref, m_s, s_s, p_s):
    j = pl.program_id(1)
    xb = x_ref[...]
    gate = jax.lax.dot_general(
        wg_ref[...], xb, (((1,), (1,)), ((), ())),
        preferred_element_type=jnp.float32)
    gm = jnp.max(gate)

    @pl.when(j == 0)
    def _init():
        e = jnp.exp(gate - gm)
        m_s[0] = gm
        s_s[0] = jnp.sum(e)
        p_s[...] = jnp.dot(e, xb, preferred_element_type=jnp.float32)

    @pl.when(j > 0)
    def _acc():
        m_old = m_s[0]
        m_new = jnp.maximum(m_old, gm)
        c = jnp.exp(m_old - m_new)
        e = jnp.exp(gate - m_new)
        m_s[0] = m_new
        s_new = s_s[0] * c + jnp.sum(e)
        s_s[0] = s_new
        p_s[...] = p_s[...] * c + jnp.dot(e, xb,
                                          preferred_element_type=jnp.float32)

    @pl.when(j == _K - 1)
    def _fin():
        s = s_s[0]
        inv = 1.0 / (s + 1e-16)
        o_ref[0] = jnp.dot(p_s[...] * inv, wn_ref[...],
                           preferred_element_type=jnp.float32) \
            + bn_ref[...] * (s * inv)


def kernel(x, Wg, bg, Wn, bn):
    bz, n, f = x.shape
    xf = x.reshape(bz * n, f)
    wgT = Wg.reshape(1, f)
    bg2 = bg.reshape(1, 1)
    bn2 = bn.reshape(1, f)
    c = n // _K
    return pl.pallas_call(
        _body,
        grid=(bz, _K),
        in_specs=[
            pl.BlockSpec((c, f), lambda b, j: (b * _K + j, 0)),
            pl.BlockSpec((1, f), lambda b, j: (0, 0)),
            pl.BlockSpec((1, 1), lambda b, j: (0, 0)),
            pl.BlockSpec((f, f), lambda b, j: (0, 0)),
            pl.BlockSpec((1, f), lambda b, j: (0, 0)),
        ],
        out_specs=pl.BlockSpec((1, 1, f), lambda b, j: (b, 0, 0)),
        out_shape=jax.ShapeDtypeStruct((bz, 1, f), jnp.float32),
        scratch_shapes=[
            pltpu.SMEM((1,), jnp.float32),
            pltpu.SMEM((1,), jnp.float32),
            pltpu.VMEM((1, f), jnp.float32),
        ],
    )(xf, wgT, bg2, Wn, bn2).reshape(bz, f)


# speedup vs baseline: 1.5732x; 1.5732x over previous
import jax
import jax.numpy as jnp
from jax.experimental import pallas as pl

_G = 2


def _body(x_ref, wg_ref, bg_ref, wn_ref, bn_ref, o_ref):
    n = x_ref.shape[0] // _G
    pooled = []
    scales = []
    for g in range(_G):
        xb = x_ref[g * n:(g + 1) * n, :]
        gate = jax.lax.dot_general(
            wg_ref[...], xb, (((1,), (1,)), ((), ())),
            preferred_element_type=jnp.float32)
        m = jnp.max(gate)
        e = jnp.exp(gate - m)
        s = jnp.sum(e)
        p = jnp.dot(e, xb, preferred_element_type=jnp.float32)
        inv = 1.0 / (s + 1e-16)
        pooled.append(p * inv)
        scales.append(s * inv)
    pcat = jnp.concatenate(pooled, axis=0)
    out = jnp.dot(pcat, wn_ref[...],
                  preferred_element_type=jnp.float32)
    for g in range(_G):
        o_ref[g] = out[g:g + 1, :] + bn_ref[...] * scales[g]


def kernel(x, Wg, bg, Wn, bn):
    bz, n, f = x.shape
    xf = x.reshape(bz * n, f)
    wgT = Wg.reshape(1, f)
    bg2 = bg.reshape(1, 1)
    bn2 = bn.reshape(1, f)
    nb = bz // _G
    return pl.pallas_call(
        _body,
        grid=(nb,),
        in_specs=[
            pl.BlockSpec((_G * n, f), lambda b: (b, 0)),
            pl.BlockSpec((1, f), lambda b: (0, 0)),
            pl.BlockSpec((1, 1), lambda b: (0, 0)),
            pl.BlockSpec((f, f), lambda b: (0, 0)),
            pl.BlockSpec((1, f), lambda b: (0, 0)),
        ],
        out_specs=pl.BlockSpec((_G, 1, f), lambda b: (b, 0, 0)),
        out_shape=jax.ShapeDtypeStruct((bz, 1, f), jnp.float32),
    )(xf, wgT, bg2, Wn, bn2).reshape(bz, f)


# speedup vs baseline: 1.6127x; 1.0251x over previous
import jax
import jax.numpy as jnp
from jax.experimental import pallas as pl

_G = 4


def _body(x_ref, wg_ref, bg_ref, wn_ref, bn_ref, o_ref):
    n = x_ref.shape[0] // _G
    pooled = []
    scales = []
    for g in range(_G):
        xb = x_ref[g * n:(g + 1) * n, :]
        gate = jax.lax.dot_general(
            wg_ref[...], xb, (((1,), (1,)), ((), ())),
            preferred_element_type=jnp.float32)
        m = jnp.max(gate)
        e = jnp.exp(gate - m)
        s = jnp.sum(e)
        p = jnp.dot(e, xb, preferred_element_type=jnp.float32)
        inv = 1.0 / (s + 1e-16)
        pooled.append(p * inv)
        scales.append(s * inv)
    pcat = jnp.concatenate(pooled, axis=0)
    out = jnp.dot(pcat, wn_ref[...],
                  preferred_element_type=jnp.float32)
    for g in range(_G):
        o_ref[g] = out[g:g + 1, :] + bn_ref[...] * scales[g]


def kernel(x, Wg, bg, Wn, bn):
    bz, n, f = x.shape
    xf = x.reshape(bz * n, f)
    wgT = Wg.reshape(1, f)
    bg2 = bg.reshape(1, 1)
    bn2 = bn.reshape(1, f)
    nb = bz // _G
    return pl.pallas_call(
        _body,
        grid=(nb,),
        in_specs=[
            pl.BlockSpec((_G * n, f), lambda b: (b, 0)),
            pl.BlockSpec((1, f), lambda b: (0, 0)),
            pl.BlockSpec((1, 1), lambda b: (0, 0)),
            pl.BlockSpec((f, f), lambda b: (0, 0)),
            pl.BlockSpec((1, f), lambda b: (0, 0)),
        ],
        out_specs=pl.BlockSpec((_G, 1, f), lambda b: (b, 0, 0)),
        out_shape=jax.ShapeDtypeStruct((bz, 1, f), jnp.float32),
    )(xf, wgT, bg2, Wn, bn2).reshape(bz, f)
